# baseline (speedup 1.0000x reference)
import functools

import jax
import jax.numpy as jnp
from jax import lax
from jax.experimental import pallas as pl
from jax.experimental.pallas import tpu as pltpu
from jax.experimental.pallas import tpu_sc as plsc

N = 50000
N_PAD = 50048
D = 64
HALF = 32
QCOL = 16
E = 800000
NS = 16
LANES = 16

ROW_W = 80
NDESC = 5
CHUNK_EDGES = NDESC * ROW_W
EROWS = E // ROW_W
ROWS_PER_TILE = EROWS // NS
N_CHUNKS = ROWS_PER_TILE // NDESC
ESLOTS = 4
ACC_ROWS_PER_TILE = N_PAD // NS


def _sc_body(dst_hbm, src_hbm, w_hbm, emb_hbm, h1_hbm, out_hbm,
             srcb, dstb, wbuf, rows, tbl, acc, esem, gsem, ssem):
    c = lax.axis_index("c")
    s = lax.axis_index("s")
    ccol = c * HALF
    coff = c * N_PAD
    wbase = s * ACC_ROWS_PER_TILE
    N_LAST = N - (NS - 1) * ACC_ROWS_PER_TILE

    def load_edges(slot, chunk):
        base = s * ROWS_PER_TILE + chunk * NDESC
        row = slot * NDESC
        pltpu.async_copy(src_hbm.at[pl.ds(base, NDESC)],
                         srcb.at[pl.ds(row, NDESC)], esem)
        pltpu.async_copy(dst_hbm.at[pl.ds(base, NDESC)],
                         dstb.at[pl.ds(row, NDESC)], esem)
        pltpu.async_copy(w_hbm.at[pl.ds(base, NDESC)],
                         wbuf.at[pl.ds(row, NDESC)], esem)

    def wait_edges():
        pltpu.make_async_copy(src_hbm.at[pl.ds(0, NDESC)],
                              srcb.at[pl.ds(0, NDESC)], esem).wait()
        pltpu.make_async_copy(dst_hbm.at[pl.ds(0, NDESC)],
                              dstb.at[pl.ds(0, NDESC)], esem).wait()
        pltpu.make_async_copy(w_hbm.at[pl.ds(0, NDESC)],
                              wbuf.at[pl.ds(0, NDESC)], esem).wait()

    def fire_gathers(slot, rbase):
        for i in range(NDESC):
            pltpu.async_copy(tbl.at[srcb.at[slot * NDESC + i]],
                             rows.at[pl.ds(rbase + i * ROW_W, ROW_W)], gsem)

    def wait_gathers():
        for i in range(NDESC):
            pltpu.make_async_copy(tbl.at[srcb.at[i]],
                                  rows.at[pl.ds(i * ROW_W, ROW_W)], gsem).wait()

    def wait_scatters():
        for i in range(NDESC):
            pltpu.make_async_copy(rows.at[pl.ds(i * ROW_W, ROW_W)],
                                  acc.at[dstb.at[i]], ssem).wait()

    def run_pass():
        load_edges(0, 0)
        wait_edges()
        fire_gathers(0, 0)
        load_edges(1, 1)

        @pl.loop(0, N_CHUNKS, unroll=1)
        def _(k):
            par = k & 1
            rbase = par * CHUNK_EDGES
            rbase_n = (1 - par) * CHUNK_EDGES
            slot = k & 3
            slot_n = (k + 1) & 3
            last = N_CHUNKS - 1

            @pl.when(k < last)
            def _():
                wait_edges()

            @pl.when(k < N_CHUNKS - 2)
            def _():
                load_edges((k + 2) & 3, k + 2)

            @pl.when(k > 0)
            def _():
                wait_scatters()

            @pl.when(k < last)
            def _():
                fire_gathers(slot_n, rbase_n)

            wait_gathers()

            for i in range(NDESC):
                for g in range(ROW_W // LANES):
                    w16 = wbuf[slot * NDESC + i, pl.ds(g * LANES, LANES)]
                    for l in range(LANES):
                        e = rbase + i * ROW_W + g * LANES + l
                        wsc = w16[l]
                        sl = pl.ds(0, LANES)
                        rows[e, sl] = rows[e, sl] * wsc
                pltpu.async_copy(rows.at[pl.ds(rbase + i * ROW_W, ROW_W)],
                                 acc.at[dstb.at[slot * NDESC + i]], ssem,
                                 add=True)

        wait_scatters()

    def zero_rows():
        @pl.loop(0, 2 * CHUNK_EDGES, unroll=4)
        def _(r):
            rows[r, pl.ds(0, LANES)] = jnp.zeros((LANES,), jnp.float32)

    def zero_acc():
        nz = 2 * CHUNK_EDGES
        for j in range(ACC_ROWS_PER_TILE // nz):
            pltpu.sync_copy(rows, acc.at[pl.ds(wbase + j * nz, nz)])
        rem = ACC_ROWS_PER_TILE % nz
        pltpu.sync_copy(rows.at[pl.ds(0, rem)],
                        acc.at[pl.ds(wbase + ACC_ROWS_PER_TILE - rem, rem)])

    def stage_emb(pcol):
        col = ccol + pcol
        for j in range(3):
            b = wbase + j * 800
            pltpu.sync_copy(emb_hbm.at[pl.ds(b, 800), pl.ds(col, QCOL)], rows)
            pltpu.sync_copy(rows, tbl.at[pl.ds(b, 800)])

        @pl.when(s < NS - 1)
        def _():
            b = wbase + 2400
            pltpu.sync_copy(emb_hbm.at[pl.ds(b, 728), pl.ds(col, QCOL)],
                            rows.at[pl.ds(0, 728)])
            pltpu.sync_copy(rows.at[pl.ds(0, 728)], tbl.at[pl.ds(b, 728)])

        @pl.when(s == NS - 1)
        def _():
            b = wbase + 2400
            pltpu.sync_copy(emb_hbm.at[pl.ds(b, 680), pl.ds(col, QCOL)],
                            rows.at[pl.ds(0, 680)])
            pltpu.sync_copy(rows.at[pl.ds(0, 680)], tbl.at[pl.ds(b, 680)])

    def stage_h1(pcol):
        for j in range(3):
            b = wbase + j * 800
            pltpu.sync_copy(h1_hbm.at[pl.ds(coff + b, 800), pl.ds(pcol, QCOL)],
                            rows)
            pltpu.sync_copy(rows, tbl.at[pl.ds(b, 800)])
        b = wbase + 2400
        pltpu.sync_copy(h1_hbm.at[pl.ds(coff + b, 728), pl.ds(pcol, QCOL)],
                        rows.at[pl.ds(0, 728)])
        pltpu.sync_copy(rows.at[pl.ds(0, 728)], tbl.at[pl.ds(b, 728)])

    def writeout_h1(pcol):
        pltpu.sync_copy(acc.at[pl.ds(wbase, ACC_ROWS_PER_TILE)],
                        h1_hbm.at[pl.ds(coff + wbase, ACC_ROWS_PER_TILE),
                                  pl.ds(pcol, QCOL)])

    def mean_part(pcol, b, sz):
        third = jnp.float32(1.0 / 3.0)
        col = ccol + pcol
        pltpu.sync_copy(emb_hbm.at[pl.ds(b, sz), pl.ds(col, QCOL)],
                        rows.at[pl.ds(0, sz)])
        pltpu.sync_copy(h1_hbm.at[pl.ds(coff + b, sz), pl.ds(pcol, QCOL)],
                        rows.at[pl.ds(256, sz)])
        pltpu.sync_copy(acc.at[pl.ds(b, sz)], rows.at[pl.ds(512, sz)])

        @pl.loop(0, sz, unroll=4)
        def _(r):
            sl = pl.ds(0, LANES)
            rows[r, sl] = (rows[r, sl] + rows[256 + r, sl]
                           + rows[512 + r, sl]) * third

        pltpu.sync_copy(rows.at[pl.ds(0, sz)],
                        out_hbm.at[pl.ds(b, sz), pl.ds(col, QCOL)])

    def mean_out(pcol):
        @pl.loop(0, 12, unroll=1)
        def _(j):
            mean_part(pcol, wbase + j * 256, 256)

        @pl.when(s < NS - 1)
        def _():
            mean_part(pcol, wbase + 3072, ACC_ROWS_PER_TILE - 3072)

        @pl.when(s == NS - 1)
        def _():
            mean_part(pcol, wbase + 3072, N_LAST - 3072)

    @pl.loop(0, 4, unroll=1)
    def _(q):
        pcol = (q & 1) * QCOL

        @pl.when(q < 2)
        def _():
            stage_emb(pcol)

        @pl.when(q >= 2)
        def _():
            stage_h1(pcol)

        zero_rows()
        zero_acc()
        plsc.subcore_barrier()
        run_pass()
        plsc.subcore_barrier()

        @pl.when(q < 2)
        def _():
            writeout_h1(pcol)

        @pl.when(q >= 2)
        def _():
            mean_out(pcol)


_propagate = functools.partial(
    pl.kernel,
    out_type=(jax.ShapeDtypeStruct((2 * N_PAD, HALF), jnp.float32),
              jax.ShapeDtypeStruct((N, D), jnp.float32)),
    mesh=plsc.VectorSubcoreMesh(core_axis_name="c", subcore_axis_name="s"),
    compiler_params=pltpu.CompilerParams(use_tc_tiling_on_sc=False),
    scratch_types=[
        pltpu.VMEM((ESLOTS * NDESC, ROW_W), jnp.int32),
        pltpu.VMEM((ESLOTS * NDESC, ROW_W), jnp.int32),
        pltpu.VMEM((ESLOTS * NDESC, ROW_W), jnp.float32),
        pltpu.VMEM((2 * CHUNK_EDGES, QCOL), jnp.float32),
        pltpu.VMEM_SHARED((N_PAD, QCOL), jnp.float32),
        pltpu.VMEM_SHARED((N_PAD, QCOL), jnp.float32),
        pltpu.SemaphoreType.DMA,
        pltpu.SemaphoreType.DMA,
        pltpu.SemaphoreType.DMA,
    ],
)(_sc_body)


def kernel(edge_index, edge_weight, emb_weight):
    dst = edge_index[0].reshape(EROWS, ROW_W)
    src = edge_index[1].reshape(EROWS, ROW_W)
    w = edge_weight.reshape(EROWS, ROW_W)
    _h1, out = _propagate(dst, src, w, emb_weight)
    return out


# speedup vs baseline: 1.4585x; 1.4585x over previous
import functools

import jax
import jax.numpy as jnp
from jax import lax
from jax.experimental import pallas as pl
from jax.experimental.pallas import tpu as pltpu
from jax.experimental.pallas import tpu_sc as plsc

N = 50000
N_PAD = 50048
D = 64
HALF = 32
E = 800000
NS = 16
LANES = 16

ROW_W = 80
NDESC = 5
CHUNK_EDGES = NDESC * ROW_W
EROWS = E // ROW_W
ROWS_PER_TILE = EROWS // NS
N_CHUNKS = ROWS_PER_TILE // NDESC
ESLOTS = 4
ACC_ROWS_PER_TILE = N_PAD // NS


def _sc_body(dst_hbm, src_hbm, w_hbm, emb_hbm, h0_hbm, h1_hbm, out_hbm,
             srcb, dstb, wbuf, rows, acc, esem, gsem, ssem):
    c = lax.axis_index("c")
    s = lax.axis_index("s")
    coff = c * N_PAD
    wbase = s * ACC_ROWS_PER_TILE
    N_LAST = N - (NS - 1) * ACC_ROWS_PER_TILE

    def load_edges(slot, chunk):
        base = s * ROWS_PER_TILE + chunk * NDESC
        row = slot * NDESC
        pltpu.async_copy(src_hbm.at[pl.ds(base, NDESC)],
                         srcb.at[pl.ds(row, NDESC)], esem)
        pltpu.async_copy(dst_hbm.at[pl.ds(base, NDESC)],
                         dstb.at[pl.ds(row, NDESC)], esem)
        pltpu.async_copy(w_hbm.at[pl.ds(base, NDESC)],
                         wbuf.at[pl.ds(row, NDESC)], esem)

    def wait_edges():
        pltpu.make_async_copy(src_hbm.at[pl.ds(0, NDESC)],
                              srcb.at[pl.ds(0, NDESC)], esem).wait()
        pltpu.make_async_copy(dst_hbm.at[pl.ds(0, NDESC)],
                              dstb.at[pl.ds(0, NDESC)], esem).wait()
        pltpu.make_async_copy(w_hbm.at[pl.ds(0, NDESC)],
                              wbuf.at[pl.ds(0, NDESC)], esem).wait()

    def prep_idx(slot):
        row = slot * NDESC
        for r in range(NDESC):
            for k in range(ROW_W // LANES):
                sl = pl.ds(k * LANES, LANES)
                srcb[row + r, sl] = srcb[row + r, sl] + coff

    def fire_gathers(h_in, slot, rbase):
        for i in range(NDESC):
            pltpu.async_copy(h_in.at[srcb.at[slot * NDESC + i]],
                             rows.at[pl.ds(rbase + i * ROW_W, ROW_W)], gsem)

    def wait_gathers(h_in):
        for i in range(NDESC):
            pltpu.make_async_copy(h_in.at[srcb.at[i]],
                                  rows.at[pl.ds(i * ROW_W, ROW_W)], gsem).wait()

    def fire_scatters(slot, rbase):
        for i in range(NDESC):
            pltpu.async_copy(rows.at[pl.ds(rbase + i * ROW_W, ROW_W)],
                             acc.at[dstb.at[slot * NDESC + i]], ssem, add=True)

    def wait_scatters():
        for i in range(NDESC):
            pltpu.make_async_copy(rows.at[pl.ds(i * ROW_W, ROW_W)],
                                  acc.at[dstb.at[i]], ssem).wait()

    def run_layer(h_in):
        load_edges(0, 0)
        wait_edges()
        prep_idx(0)
        fire_gathers(h_in, 0, 0)
        load_edges(1, 1)

        @pl.loop(0, N_CHUNKS, unroll=1)
        def _(k):
            par = k & 1
            rbase = par * CHUNK_EDGES
            rbase_n = (1 - par) * CHUNK_EDGES
            slot = k & 3
            slot_n = (k + 1) & 3
            last = N_CHUNKS - 1

            @pl.when(k < last)
            def _():
                wait_edges()
                prep_idx(slot_n)

            @pl.when(k < N_CHUNKS - 2)
            def _():
                load_edges((k + 2) & 3, k + 2)

            @pl.when(k > 0)
            def _():
                wait_scatters()

            @pl.when(k < last)
            def _():
                fire_gathers(h_in, slot_n, rbase_n)

            wait_gathers(h_in)

            for i in range(NDESC):
                for g in range(ROW_W // LANES):
                    w16 = wbuf[slot * NDESC + i, pl.ds(g * LANES, LANES)]
                    for l in range(LANES):
                        e = rbase + i * ROW_W + g * LANES + l
                        wsc = w16[l]
                        for kk in range(HALF // LANES):
                            sl = pl.ds(kk * LANES, LANES)
                            rows[e, sl] = rows[e, sl] * wsc
                pltpu.async_copy(rows.at[pl.ds(rbase + i * ROW_W, ROW_W)],
                                 acc.at[dstb.at[slot * NDESC + i]], ssem,
                                 add=True)

        wait_scatters()

    def zero_rows():
        @pl.loop(0, 2 * CHUNK_EDGES, unroll=4)
        def _(r):
            for k in range(HALF // LANES):
                rows[r, pl.ds(k * LANES, LANES)] = jnp.zeros((LANES,), jnp.float32)

    def zero_acc():
        nz = 2 * CHUNK_EDGES
        for j in range(ACC_ROWS_PER_TILE // nz):
            pltpu.sync_copy(rows, acc.at[pl.ds(wbase + j * nz, nz)])
        rem = ACC_ROWS_PER_TILE % nz
        pltpu.sync_copy(rows.at[pl.ds(0, rem)],
                        acc.at[pl.ds(wbase + ACC_ROWS_PER_TILE - rem, rem)])

    def writeout(h_out):
        pltpu.sync_copy(acc.at[pl.ds(wbase, ACC_ROWS_PER_TILE)],
                        h_out.at[pl.ds(coff + wbase, ACC_ROWS_PER_TILE)])

    def build_h0_part(b, sz):
        pltpu.sync_copy(emb_hbm.at[pl.ds(b, sz), pl.ds(c * HALF, HALF)],
                        rows.at[pl.ds(0, sz)])
        pltpu.sync_copy(rows.at[pl.ds(0, sz)], h0_hbm.at[pl.ds(coff + b, sz)])

    def build_h0():
        for j in range(4):
            build_h0_part(wbase + j * 768, 768)

        @pl.when(s < NS - 1)
        def _():
            build_h0_part(wbase + 3072, ACC_ROWS_PER_TILE - 3072)

        @pl.when(s == NS - 1)
        def _():
            build_h0_part(wbase + 3072, N_LAST - 3072)

    def mean_part(b, sz):
        third = jnp.float32(1.0 / 3.0)
        pltpu.sync_copy(emb_hbm.at[pl.ds(b, sz), pl.ds(c * HALF, HALF)],
                        rows.at[pl.ds(0, sz)])
        pltpu.sync_copy(h1_hbm.at[pl.ds(coff + b, sz)], rows.at[pl.ds(256, sz)])
        pltpu.sync_copy(acc.at[pl.ds(b, sz)], rows.at[pl.ds(512, sz)])

        @pl.loop(0, sz, unroll=4)
        def _(r):
            for k2 in range(HALF // LANES):
                sl = pl.ds(k2 * LANES, LANES)
                rows[r, sl] = (rows[r, sl] + rows[256 + r, sl]
                               + rows[512 + r, sl]) * third

        pltpu.sync_copy(rows.at[pl.ds(0, sz)],
                        out_hbm.at[pl.ds(b, sz), pl.ds(c * HALF, HALF)])

    def mean_out():
        @pl.loop(0, 12, unroll=1)
        def _(j):
            mean_part(wbase + j * 256, 256)

        @pl.when(s < NS - 1)
        def _():
            mean_part(wbase + 3072, ACC_ROWS_PER_TILE - 3072)

        @pl.when(s == NS - 1)
        def _():
            mean_part(wbase + 3072, N_LAST - 3072)

    build_h0()
    zero_rows()
    zero_acc()
    plsc.subcore_barrier()
    run_layer(h0_hbm)
    plsc.subcore_barrier()
    writeout(h1_hbm)
    zero_rows()
    zero_acc()
    plsc.subcore_barrier()
    run_layer(h1_hbm)
    plsc.subcore_barrier()
    mean_out()


_propagate = functools.partial(
    pl.kernel,
    out_type=(jax.ShapeDtypeStruct((2 * N_PAD, HALF), jnp.float32),
              jax.ShapeDtypeStruct((2 * N_PAD, HALF), jnp.float32),
              jax.ShapeDtypeStruct((N, D), jnp.float32)),
    mesh=plsc.VectorSubcoreMesh(core_axis_name="c", subcore_axis_name="s"),
    compiler_params=pltpu.CompilerParams(use_tc_tiling_on_sc=False),
    scratch_types=[
        pltpu.VMEM((ESLOTS * NDESC, ROW_W), jnp.int32),
        pltpu.VMEM((ESLOTS * NDESC, ROW_W), jnp.int32),
        pltpu.VMEM((ESLOTS * NDESC, ROW_W), jnp.float32),
        pltpu.VMEM((2 * CHUNK_EDGES, HALF), jnp.float32),
        pltpu.VMEM_SHARED((N_PAD, HALF), jnp.float32),
        pltpu.SemaphoreType.DMA,
        pltpu.SemaphoreType.DMA,
        pltpu.SemaphoreType.DMA,
    ],
)(_sc_body)


def kernel(edge_index, edge_weight, emb_weight):
    dst = edge_index[0].reshape(EROWS, ROW_W)
    src = edge_index[1].reshape(EROWS, ROW_W)
    w = edge_weight.reshape(EROWS, ROW_W)
    _h0, _h1, out = _propagate(dst, src, w, emb_weight)
    return out


# speedup vs baseline: 1.4590x; 1.0003x over previous
import functools

import jax
import jax.numpy as jnp
from jax import lax
from jax.experimental import pallas as pl
from jax.experimental.pallas import tpu as pltpu
from jax.experimental.pallas import tpu_sc as plsc

N = 50000
N_PAD = 50048
D = 64
HALF = 32
E = 800000
NS = 16
LANES = 16

ROW_W = 80
NDESC = 5
CHUNK_EDGES = NDESC * ROW_W
EROWS = E // ROW_W
ROWS_PER_TILE = EROWS // NS
N_CHUNKS = ROWS_PER_TILE // NDESC
ESLOTS = 4
ACC_ROWS_PER_TILE = N_PAD // NS


def _sc_body(dst_hbm, src_hbm, w_hbm, emb_hbm, h0_hbm, h1_hbm, out_hbm,
             srcb, dstb, wbuf, rows, acc, esem, gsem, ssem):
    c = lax.axis_index("c")
    s = lax.axis_index("s")
    coff = c * N_PAD
    wbase = s * ACC_ROWS_PER_TILE
    N_LAST = N - (NS - 1) * ACC_ROWS_PER_TILE

    def load_edges(slot, chunk):
        base = s * ROWS_PER_TILE + chunk * NDESC
        row = slot * NDESC
        pltpu.async_copy(src_hbm.at[pl.ds(base, NDESC)],
                         srcb.at[pl.ds(row, NDESC)], esem)
        pltpu.async_copy(dst_hbm.at[pl.ds(base, NDESC)],
                         dstb.at[pl.ds(row, NDESC)], esem)
        pltpu.async_copy(w_hbm.at[pl.ds(base, NDESC)],
                         wbuf.at[pl.ds(row, NDESC)], esem)

    def wait_edges():
        pltpu.make_async_copy(src_hbm.at[pl.ds(0, NDESC)],
                              srcb.at[pl.ds(0, NDESC)], esem).wait()
        pltpu.make_async_copy(dst_hbm.at[pl.ds(0, NDESC)],
                              dstb.at[pl.ds(0, NDESC)], esem).wait()
        pltpu.make_async_copy(w_hbm.at[pl.ds(0, NDESC)],
                              wbuf.at[pl.ds(0, NDESC)], esem).wait()

    def prep_idx(slot):
        row = slot * NDESC
        for r in range(NDESC):
            for k in range(ROW_W // LANES):
                sl = pl.ds(k * LANES, LANES)
                srcb[row + r, sl] = srcb[row + r, sl] + coff

    def fire_gathers(h_in, slot, rbase):
        for i in range(NDESC):
            pltpu.async_copy(h_in.at[srcb.at[slot * NDESC + i]],
                             rows.at[pl.ds(rbase + i * ROW_W, ROW_W)], gsem)

    def wait_gathers(h_in):
        for i in range(NDESC):
            pltpu.make_async_copy(h_in.at[srcb.at[i]],
                                  rows.at[pl.ds(i * ROW_W, ROW_W)], gsem).wait()

    def fire_scatters(slot, rbase):
        for i in range(NDESC):
            pltpu.async_copy(rows.at[pl.ds(rbase + i * ROW_W, ROW_W)],
                             acc.at[dstb.at[slot * NDESC + i]], ssem, add=True)

    def wait_scatters():
        for i in range(NDESC):
            pltpu.make_async_copy(rows.at[pl.ds(i * ROW_W, ROW_W)],
                                  acc.at[dstb.at[i]], ssem).wait()

    def run_layer(h_in):
        load_edges(0, 0)
        wait_edges()
        prep_idx(0)
        fire_gathers(h_in, 0, 0)
        load_edges(1, 1)

        @pl.loop(0, N_CHUNKS, unroll=1)
        def _(k):
            par = k & 1
            rbase = par * CHUNK_EDGES
            rbase_n = (1 - par) * CHUNK_EDGES
            slot = k & 3
            slot_n = (k + 1) & 3
            last = N_CHUNKS - 1

            @pl.when(k < last)
            def _():
                wait_edges()
                prep_idx(slot_n)

            @pl.when(k < N_CHUNKS - 2)
            def _():
                load_edges((k + 2) & 3, k + 2)

            @pl.when(k > 0)
            def _():
                wait_scatters()

            @pl.when(k < last)
            def _():
                fire_gathers(h_in, slot_n, rbase_n)

            wait_gathers(h_in)

            for i in range(NDESC):
                for g in range(ROW_W // LANES):
                    w16 = wbuf[slot * NDESC + i, pl.ds(g * LANES, LANES)]
                    for l in range(LANES):
                        e = rbase + i * ROW_W + g * LANES + l
                        wsc = w16[l]
                        for kk in range(HALF // LANES):
                            sl = pl.ds(kk * LANES, LANES)
                            rows[e, sl] = rows[e, sl] * wsc
                pltpu.async_copy(rows.at[pl.ds(rbase + i * ROW_W, ROW_W)],
                                 acc.at[dstb.at[slot * NDESC + i]], ssem,
                                 add=True)

        wait_scatters()

    def zero_rows():
        @pl.loop(0, 2 * CHUNK_EDGES, unroll=4)
        def _(r):
            for k in range(HALF // LANES):
                rows[r, pl.ds(k * LANES, LANES)] = jnp.zeros((LANES,), jnp.float32)

    def zero_acc():
        nz = 2 * CHUNK_EDGES
        for j in range(ACC_ROWS_PER_TILE // nz):
            pltpu.sync_copy(rows, acc.at[pl.ds(wbase + j * nz, nz)])
        rem = ACC_ROWS_PER_TILE % nz
        pltpu.sync_copy(rows.at[pl.ds(0, rem)],
                        acc.at[pl.ds(wbase + ACC_ROWS_PER_TILE - rem, rem)])

    def writeout(h_out):
        pltpu.sync_copy(acc.at[pl.ds(wbase, ACC_ROWS_PER_TILE)],
                        h_out.at[pl.ds(coff + wbase, ACC_ROWS_PER_TILE)])

    def build_h0_part(b, sz):
        pltpu.sync_copy(emb_hbm.at[pl.ds(b, sz), pl.ds(c * HALF, HALF)],
                        rows.at[pl.ds(0, sz)])
        pltpu.sync_copy(rows.at[pl.ds(0, sz)], h0_hbm.at[pl.ds(coff + b, sz)])

    def build_h0():
        for j in range(4):
            build_h0_part(wbase + j * 768, 768)

        @pl.when(s < NS - 1)
        def _():
            build_h0_part(wbase + 3072, ACC_ROWS_PER_TILE - 3072)

        @pl.when(s == NS - 1)
        def _():
            build_h0_part(wbase + 3072, N_LAST - 3072)

    def mean_part(b, sz):
        third = jnp.float32(1.0 / 3.0)
        pltpu.sync_copy(emb_hbm.at[pl.ds(b, sz), pl.ds(c * HALF, HALF)],
                        rows.at[pl.ds(0, sz)])
        pltpu.sync_copy(h1_hbm.at[pl.ds(coff + b, sz)], rows.at[pl.ds(256, sz)])
        pltpu.sync_copy(acc.at[pl.ds(b, sz)], rows.at[pl.ds(512, sz)])

        @pl.loop(0, sz, unroll=4)
        def _(r):
            for k2 in range(HALF // LANES):
                sl = pl.ds(k2 * LANES, LANES)
                rows[r, sl] = (rows[r, sl] + rows[256 + r, sl]
                               + rows[512 + r, sl]) * third

        pltpu.sync_copy(rows.at[pl.ds(0, sz)],
                        out_hbm.at[pl.ds(b, sz), pl.ds(c * HALF, HALF)])

    def mean_out():
        @pl.loop(0, 12, unroll=1)
        def _(j):
            mean_part(wbase + j * 256, 256)

        @pl.when(s < NS - 1)
        def _():
            mean_part(wbase + 3072, ACC_ROWS_PER_TILE - 3072)

        @pl.when(s == NS - 1)
        def _():
            mean_part(wbase + 3072, N_LAST - 3072)

    build_h0()
    zero_rows()
    zero_acc()
    plsc.subcore_barrier()
    run_layer(h0_hbm)
    plsc.subcore_barrier()
    writeout(h1_hbm)
    zero_rows()
    zero_acc()
    plsc.subcore_barrier()
    run_layer(h1_hbm)
    plsc.subcore_barrier()
    mean_out()


_propagate = functools.partial(
    pl.kernel,
    out_type=(jax.ShapeDtypeStruct((2 * N_PAD, HALF), jnp.float32),
              jax.ShapeDtypeStruct((2 * N_PAD, HALF), jnp.float32),
              jax.ShapeDtypeStruct((N, D), jnp.float32)),
    mesh=plsc.VectorSubcoreMesh(core_axis_name="c", subcore_axis_name="s"),
    compiler_params=pltpu.CompilerParams(use_tc_tiling_on_sc=False,
                                         skip_device_barrier=True),
    scratch_types=[
        pltpu.VMEM((ESLOTS * NDESC, ROW_W), jnp.int32),
        pltpu.VMEM((ESLOTS * NDESC, ROW_W), jnp.int32),
        pltpu.VMEM((ESLOTS * NDESC, ROW_W), jnp.float32),
        pltpu.VMEM((2 * CHUNK_EDGES, HALF), jnp.float32),
        pltpu.VMEM_SHARED((N_PAD, HALF), jnp.float32),
        pltpu.SemaphoreType.DMA,
        pltpu.SemaphoreType.DMA,
        pltpu.SemaphoreType.DMA,
    ],
)(_sc_body)


def kernel(edge_index, edge_weight, emb_weight):
    dst = edge_index[0].reshape(EROWS, ROW_W)
    src = edge_index[1].reshape(EROWS, ROW_W)
    w = edge_weight.reshape(EROWS, ROW_W)
    _h0, _h1, out = _propagate(dst, src, w, emb_weight)
    return out
